# baseline (speedup 1.0000x reference)
import functools

import jax
import jax.numpy as jnp
from jax import lax
from jax.experimental import pallas as pl
from jax.experimental.pallas import tpu as pltpu
from jax.experimental.pallas import tpu_sc as plsc

NC = 2
NS = 16
L = 16


def _linear_kernel(x_ref, w_ref, b_ref, h_ref):
    h_ref[...] = jnp.dot(
        x_ref[...], w_ref[...], preferred_element_type=jnp.float32
    ) + b_ref[...]


def _combine_kernel(p0_ref, p1_ref, o_ref):
    o_ref[...] = jnp.maximum(p0_ref[...] + p1_ref[...], 0.0)


def _make_sc_kernel(n_pad, d, e, k):
    epw = e // (NC * NS)
    n_chunks = epw // k
    assert n_chunks >= 4
    rpw = n_pad // NS

    mesh = plsc.VectorSubcoreMesh(core_axis_name="c", subcore_axis_name="s")

    @functools.partial(
        pl.kernel,
        out_type=(
            jax.ShapeDtypeStruct((n_pad, d), jnp.float32),
            jax.ShapeDtypeStruct((n_pad, d), jnp.float32),
        ),
        mesh=mesh,
        compiler_params=pltpu.CompilerParams(needs_layout_passes=False),
        scratch_types=[
            pltpu.VMEM_SHARED((n_pad, d), jnp.float32),
            [pltpu.VMEM((k,), jnp.int32)] * 2,
            [pltpu.VMEM((2, k // 2), jnp.int32)] * 2,
            [pltpu.VMEM((k + 8,), jnp.float32)] * 2,
            [pltpu.VMEM((k, d), jnp.float32)] * 2,
            [pltpu.SemaphoreType.DMA] * 2,
            [pltpu.SemaphoreType.DMA] * 2,
            [pltpu.SemaphoreType.DMA] * 2,
            [pltpu.SemaphoreType.DMA] * 2,
        ],
    )
    def sc_kernel(h, src, dst, ev, zrows, p0, p1,
                  acc, src_v, dst_v, ev_v, rows_v, isem, gsem, dsem, ssem):
        c = lax.axis_index("c")
        s = lax.axis_index("s")

        pltpu.sync_copy(zrows, acc.at[pl.ds(s * rpw, rpw)])
        plsc.subcore_barrier()

        base0 = (c * NS + s) * epw
        last = n_chunks - 1
        k2 = k // 2

        def clamp(ci):
            return jnp.minimum(ci, last)

        def gstart(j):
            for u in range(2):
                pltpu.async_copy(h.at[src_v[j].at[pl.ds(u * k2, k2)]],
                                 rows_v[j].at[pl.ds(u * k2, k2)], gsem[j])

        def gwait(j):
            for u in range(2):
                pltpu.make_async_copy(h.at[src_v[j].at[pl.ds(u * k2, k2)]],
                                      rows_v[j].at[pl.ds(u * k2, k2)],
                                      gsem[j]).wait()

        def sstart(j):
            for u in range(2):
                pltpu.async_copy(rows_v[j].at[pl.ds(u * k2, k2)],
                                 acc.at[dst_v[j].at[u]], ssem[j], add=True)

        def swait(j):
            for u in range(2):
                pltpu.make_async_copy(rows_v[j].at[pl.ds(u * k2, k2)],
                                      acc.at[dst_v[j].at[u]],
                                      ssem[j]).wait()

        def istart(ci, j):
            base = base0 + clamp(ci) * k
            pltpu.async_copy(src.at[pl.ds(base, k)], src_v[j], isem[j])
            pltpu.async_copy(
                ev.at[pl.ds(base, k)], ev_v[j].at[pl.ds(8, k)], isem[j])

        def iwait(j):
            pltpu.make_async_copy(src.at[pl.ds(0, k)], src_v[j],
                                  isem[j]).wait()
            pltpu.make_async_copy(ev.at[pl.ds(0, k)],
                                  ev_v[j].at[pl.ds(8, k)], isem[j]).wait()

        def half(ci, j, first):
            o = 1 - j
            base = base0 + clamp(ci) * k
            for u in range(2):
                pltpu.async_copy(dst.at[pl.ds(base + u * k2, k2)],
                                 dst_v[j].at[u], dsem[j])
            if not first:
                swait(o)
            iwait(o)
            gstart(o)
            gwait(j)
            for g in range(k // L):
                evg = ev_v[j][pl.ds(8 + g * L, L)]
                for t in range(L):
                    ei = g * L + t
                    scale = evg.at[jnp.full((L,), t, jnp.int32)].get(
                        mode="promise_in_bounds")
                    for dv in range(d // L):
                        sl = pl.ds(dv * L, L)
                        rows_v[j][ei, sl] = rows_v[j][ei, sl] * scale
            for u in range(2):
                pltpu.make_async_copy(dst.at[pl.ds(0, k2)], dst_v[j].at[u],
                                      dsem[j]).wait()
            sstart(j)
            istart(ci + 2, j)

        istart(0, 0)
        iwait(0)
        gstart(0)
        istart(1, 1)

        half(0, 0, True)
        half(1, 1, False)

        @pl.loop(1, n_chunks // 2)
        def _pairs(p):
            a = 2 * p
            half(a, 0, False)
            half(a + 1, 1, False)

        if n_chunks % 2 == 1:
            half(last, 0, False)

        fp = last % 2
        swait(fp)
        gwait(1 - fp)
        iwait(fp)

        plsc.subcore_barrier()

        rows = pl.ds(s * rpw, rpw)

        @pl.when(c == 0)
        def _():
            pltpu.sync_copy(acc.at[rows], p0.at[rows])

        @pl.when(c == 1)
        def _():
            pltpu.sync_copy(acc.at[rows], p1.at[rows])

    return sc_kernel


@jax.jit
def kernel(x, edge_index, edge_vals, W, b):
    n, d = x.shape
    e = edge_vals.shape[0]
    k = 128
    n_pad = ((n + NS * 8 - 1) // (NS * 8)) * (NS * 8)
    e_pad = ((e + NC * NS * k - 1) // (NC * NS * k)) * (NC * NS * k)
    assert d % L == 0

    h = pl.pallas_call(
        _linear_kernel,
        out_shape=jax.ShapeDtypeStruct((n, d), jnp.float32),
    )(x, W, b.reshape(1, d))

    pe = e_pad - e
    src = jnp.pad(edge_index[0], (0, pe))
    dst = jnp.pad(edge_index[1], (0, pe))
    ev = jnp.pad(edge_vals, (0, pe))
    zrows = jnp.zeros((n_pad // NS, d), jnp.float32)

    sc = _make_sc_kernel(n_pad, d, e_pad, k)
    p0, p1 = sc(h, src, dst, ev, zrows)

    out = pl.pallas_call(
        _combine_kernel,
        out_shape=jax.ShapeDtypeStruct((n_pad, d), jnp.float32),
    )(p0, p1)
    return out[:n]


# speedup vs baseline: 1.7817x; 1.7817x over previous
import functools

import jax
import jax.numpy as jnp
from jax import lax
from jax.experimental import pallas as pl
from jax.experimental.pallas import tpu as pltpu
from jax.experimental.pallas import tpu_sc as plsc

NC = 2
NS = 16
L = 16


def _linear_kernel(x_ref, w_ref, b_ref, h_ref):
    h_ref[...] = jnp.dot(
        x_ref[...], w_ref[...], preferred_element_type=jnp.float32
    ) + b_ref[...]


def _combine_kernel(p0_ref, p1_ref, o_ref):
    o_ref[...] = jnp.maximum(p0_ref[...] + p1_ref[...], 0.0)


def _make_sc_kernel(n_pad, d, e, k):
    epw = e // (NC * NS)
    n_chunks = epw // k
    assert n_chunks >= 4
    rpw = n_pad // NS

    mesh = plsc.VectorSubcoreMesh(core_axis_name="c", subcore_axis_name="s")

    @functools.partial(
        pl.kernel,
        out_type=(
            jax.ShapeDtypeStruct((n_pad, d), jnp.float32),
            jax.ShapeDtypeStruct((n_pad, d), jnp.float32),
        ),
        mesh=mesh,
        compiler_params=pltpu.CompilerParams(needs_layout_passes=False),
        scratch_types=[
            pltpu.VMEM_SHARED((n_pad, d), jnp.float32),
            [pltpu.VMEM((k,), jnp.int32)] * 2,
            [pltpu.VMEM((2, k // 2), jnp.int32)] * 2,
            [pltpu.VMEM((k + 8,), jnp.float32)] * 2,
            [pltpu.VMEM((k, d), jnp.float32)] * 2,
            [pltpu.SemaphoreType.DMA] * 2,
            [pltpu.SemaphoreType.DMA] * 2,
            [pltpu.SemaphoreType.DMA] * 2,
            [pltpu.SemaphoreType.DMA] * 2,
        ],
    )
    def sc_kernel(h, src, dst, ev, zrows, p0, p1,
                  acc, src_v, dst_v, ev_v, rows_v, isem, gsem, dsem, ssem):
        c = lax.axis_index("c")
        s = lax.axis_index("s")

        pltpu.sync_copy(zrows, acc.at[pl.ds(s * rpw, rpw)])
        plsc.subcore_barrier()

        base0 = (c * NS + s) * epw
        last = n_chunks - 1
        k2 = k // 2

        def clamp(ci):
            return jnp.minimum(ci, last)

        def gstart(j):
            for u in range(2):
                pltpu.async_copy(h.at[src_v[j].at[pl.ds(u * k2, k2)]],
                                 rows_v[j].at[pl.ds(u * k2, k2)], gsem[j])

        def gwait(j):
            for u in range(2):
                pltpu.make_async_copy(h.at[src_v[j].at[pl.ds(u * k2, k2)]],
                                      rows_v[j].at[pl.ds(u * k2, k2)],
                                      gsem[j]).wait()

        def sstart(j):
            for u in range(2):
                pltpu.async_copy(rows_v[j].at[pl.ds(u * k2, k2)],
                                 acc.at[dst_v[j].at[u]], ssem[j], add=True)

        def swait(j):
            for u in range(2):
                pltpu.make_async_copy(rows_v[j].at[pl.ds(u * k2, k2)],
                                      acc.at[dst_v[j].at[u]],
                                      ssem[j]).wait()

        def istart(ci, j):
            base = base0 + clamp(ci) * k
            pltpu.async_copy(src.at[pl.ds(base, k)], src_v[j], isem[j])
            pltpu.async_copy(
                ev.at[pl.ds(base, k)], ev_v[j].at[pl.ds(8, k)], isem[j])

        def iwait(j):
            pltpu.make_async_copy(src.at[pl.ds(0, k)], src_v[j],
                                  isem[j]).wait()
            pltpu.make_async_copy(ev.at[pl.ds(0, k)],
                                  ev_v[j].at[pl.ds(8, k)], isem[j]).wait()

        def half(ci, j, first):
            o = 1 - j
            base = base0 + clamp(ci) * k
            for u in range(2):
                pltpu.async_copy(dst.at[pl.ds(base + u * k2, k2)],
                                 dst_v[j].at[u], dsem[j])
            if not first:
                swait(o)
            iwait(o)
            gstart(o)
            gwait(j)
            for g in range(k // L):
                evg = ev_v[j][pl.ds(8 + g * L, L)]
                for t in range(L):
                    ei = g * L + t
                    scale = evg.at[jnp.full((L,), t, jnp.int32)].get(
                        mode="promise_in_bounds")
                    for dv in range(d // L):
                        sl = pl.ds(dv * L, L)
                        rows_v[j][ei, sl] = rows_v[j][ei, sl] * scale
            for u in range(2):
                pltpu.make_async_copy(dst.at[pl.ds(0, k2)], dst_v[j].at[u],
                                      dsem[j]).wait()
            sstart(j)
            istart(ci + 2, j)

        istart(0, 0)
        iwait(0)
        gstart(0)
        istart(1, 1)

        half(0, 0, True)
        half(1, 1, False)

        @pl.loop(1, n_chunks // 2)
        def _pairs(p):
            a = 2 * p
            half(a, 0, False)
            half(a + 1, 1, False)

        if n_chunks % 2 == 1:
            half(last, 0, False)

        fp = last % 2
        swait(fp)
        gwait(1 - fp)
        iwait(fp)

        plsc.subcore_barrier()

        rows = pl.ds(s * rpw, rpw)

        @pl.when(c == 0)
        def _():
            pltpu.sync_copy(acc.at[rows], p0.at[rows])

        @pl.when(c == 1)
        def _():
            pltpu.sync_copy(acc.at[rows], p1.at[rows])

    return sc_kernel


@jax.jit
def kernel(x, edge_index, edge_vals, W, b):
    n, d = x.shape
    e = edge_vals.shape[0]
    k = 80
    n_pad = ((n + NS * 8 - 1) // (NS * 8)) * (NS * 8)
    e_pad = ((e + NC * NS * k - 1) // (NC * NS * k)) * (NC * NS * k)
    assert d % L == 0

    h = pl.pallas_call(
        _linear_kernel,
        out_shape=jax.ShapeDtypeStruct((n, d), jnp.float32),
    )(x, W, b.reshape(1, d))

    pe = e_pad - e
    src = jnp.pad(edge_index[0], (0, pe))
    dst = jnp.pad(edge_index[1], (0, pe))
    ev = jnp.pad(edge_vals, (0, pe))
    zrows = jnp.zeros((n_pad // NS, d), jnp.float32)

    sc = _make_sc_kernel(n_pad, d, e_pad, k)
    p0, p1 = sc(h, src, dst, ev, zrows)

    out = pl.pallas_call(
        _combine_kernel,
        out_shape=jax.ShapeDtypeStruct((n_pad, d), jnp.float32),
    )(p0, p1)
    return out[:n]
